# Initial kernel scaffold; baseline (speedup 1.0000x reference)
import jax
import jax.numpy as jnp
from jax.experimental import pallas as pl


def kernel(reg_degree, gate_is_input, dev_edge_index, circ_edge_index, qubit_physical_idx, edge_reg_indices, reg_table, gate_table, W_self0, W_neigh0, b_sage0, W_self1, W_neigh1, b_sage1, W_circ0, b_circ0, W_circ1, b_circ1):
    raise NotImplementedError("write your pallas kernel here")


# speedup vs baseline: 2.6174x; 2.6174x over previous
import functools

import jax
import jax.numpy as jnp
from jax import lax
from jax.experimental import pallas as pl
from jax.experimental.pallas import tpu as pltpu
from jax.experimental.pallas import tpu_sc as plsc

NR = 50000
NG = 50000
ED = 800000
EC = 800000
NQ = 4096
DEG_TYPES = 16

NP_ = 53248
EDP = 819200
BLK = 1024
NBLK = NP_ // BLK
TSTRIPE = NP_ // 16
ECH = 2048
NCHUNK = (EDP // 16) // ECH
SUB = 4
CCH = 2048
NCCHUNK = (EDP // 16) // CCH

_MESH = plsc.VectorSubcoreMesh(core_axis_name="c", subcore_axis_name="s")


def _leaky(x):
    return jnp.where(x >= 0, x, 0.01 * x)


def _embed_body(deg_ref, gii_ref, rtab_ref, gtab_ref, remb_ref, gemb_ref):
    i = pl.program_id(0)
    deg = deg_ref[...]
    iota16 = lax.broadcasted_iota(jnp.int32, (BLK, DEG_TYPES), 1)
    onehot = (deg == iota16).astype(jnp.float32)
    remb_ref[...] = jnp.dot(onehot, rtab_ref[...],
                            preferred_element_type=jnp.float32)
    row0 = gtab_ref[0:1, :]
    row1 = gtab_ref[1:2, :]
    gemb = jnp.where(gii_ref[...] == 1, row1, row0)
    rid = i * BLK + lax.broadcasted_iota(jnp.int32, (BLK, 1), 0)
    gemb_ref[...] = jnp.where(rid < NG, gemb, 0.0)


def _embed(deg2d, gii2d, rtab, gtab):
    return pl.pallas_call(
        _embed_body,
        grid=(NBLK,),
        in_specs=[
            pl.BlockSpec((BLK, 1), lambda i: (i, 0)),
            pl.BlockSpec((BLK, 1), lambda i: (i, 0)),
            pl.BlockSpec((DEG_TYPES, 32), lambda i: (0, 0)),
            pl.BlockSpec((2, 32), lambda i: (0, 0)),
        ],
        out_specs=[
            pl.BlockSpec((BLK, 32), lambda i: (i, 0)),
            pl.BlockSpec((BLK, 32), lambda i: (i, 0)),
        ],
        out_shape=[
            jax.ShapeDtypeStruct((NP_, 32), jnp.float32),
            jax.ShapeDtypeStruct((NP_, 32), jnp.float32),
        ],
    )(deg2d, gii2d, rtab, gtab)


def _prep_body(d, re_ref, ma_ref, ge_ref, ws_ref, wn_ref, wc_ref,
               s_ref, plo_ref, phi_ref, glo_ref, ghi_ref, elo_ref, ehi_ref):
    re = re_ref[...]
    ma = ma_ref[...]
    ge = ge_ref[...]
    ws = ws_ref[...]
    wn = wn_ref[...]
    wc = wc_ref[...]
    dot = functools.partial(jnp.dot, preferred_element_type=jnp.float32)
    s = dot(re, ws[:d]) + dot(ma, ws[d:])
    p = dot(re, wn[:d]) + dot(ma, wn[d:])
    g = dot(ge, wc[:d])
    e = dot(re, wc[d:])
    s_ref[...] = s
    plo_ref[...] = p[:, :32]
    phi_ref[...] = p[:, 32:]
    glo_ref[...] = g[:, :32]
    ghi_ref[...] = g[:, 32:]
    elo_ref[...] = e[:, :32]
    ehi_ref[...] = e[:, 32:]


def _prep(d, reg_emb, matched, gate_emb, ws, wn, wc):
    node64 = lambda: pl.BlockSpec((BLK, 64), lambda i: (i, 0))
    node32 = lambda: pl.BlockSpec((BLK, 32), lambda i: (i, 0))
    noded = lambda: pl.BlockSpec((BLK, d), lambda i: (i, 0))
    w2d = lambda: pl.BlockSpec((2 * d, 64), lambda i: (0, 0))
    return pl.pallas_call(
        functools.partial(_prep_body, d),
        grid=(NBLK,),
        in_specs=[noded(), noded(), noded(), w2d(), w2d(), w2d()],
        out_specs=[node64()] + [node32()] * 6,
        out_shape=[jax.ShapeDtypeStruct((NP_, 64), jnp.float32)]
        + [jax.ShapeDtypeStruct((NP_, 32), jnp.float32)] * 6,
    )(reg_emb, matched, gate_emb, ws, wn, wc)


def _epilogue_body(s_ref, alo_ref, ahi_ref, cntd_ref, bs_ref,
                   blo_ref, bhi_ref, cntc_ref, bc_ref,
                   nreg_ref, ngate_ref):
    inv_d = 1.0 / jnp.maximum(cntd_ref[:, 0:1], 1.0)
    cnt_c = cntc_ref[:, 0:1]
    inv_c = 1.0 / jnp.maximum(cnt_c, 1.0)
    gate_on = (cnt_c > 0).astype(jnp.float32)
    s = s_ref[...]
    bs = bs_ref[...]
    bc = bc_ref[...]
    nreg_ref[:, 0:32] = _leaky(s[:, 0:32] + alo_ref[...] * inv_d + bs[:, 0:32])
    nreg_ref[:, 32:64] = _leaky(s[:, 32:64] + ahi_ref[...] * inv_d + bs[:, 32:64])
    ngate_ref[:, 0:32] = _leaky(blo_ref[...] * inv_c + bc[:, 0:32] * gate_on)
    ngate_ref[:, 32:64] = _leaky(bhi_ref[...] * inv_c + bc[:, 32:64] * gate_on)


def _epilogue(s, a_lo, a_hi, cnt_d, bs2d, b_lo, b_hi, cnt_c, bc2d):
    node64 = lambda: pl.BlockSpec((BLK, 64), lambda i: (i, 0))
    node32 = lambda: pl.BlockSpec((BLK, 32), lambda i: (i, 0))
    cnt = lambda: pl.BlockSpec((BLK, 16), lambda i: (i, 0))
    bias = lambda: pl.BlockSpec((1, 64), lambda i: (0, 0))
    return pl.pallas_call(
        _epilogue_body,
        grid=(NBLK,),
        in_specs=[node64(), node32(), node32(), cnt(), bias(),
                  node32(), node32(), cnt(), bias()],
        out_specs=[node64(), node64()],
        out_shape=[jax.ShapeDtypeStruct((NP_, 64), jnp.float32)] * 2,
    )(s, a_lo, a_hi, cnt_d, bs2d, b_lo, b_hi, cnt_c, bc2d)


_NGROUP = NP_ // 1024


def _gather_rows_body(tbl_ref, win2d_ref, out_ref, idx_v, rows_v, sem):
    c = lax.axis_index("c")
    s = lax.axis_index("s")
    wid = s * 2 + c
    for rep in range((_NGROUP + 31) // 32):
        g = rep * 32 + wid

        @pl.when(g < _NGROUP)
        def _():
            row0 = pl.multiple_of(g * 8, 8)
            pltpu.sync_copy(win2d_ref.at[pl.ds(row0, 8)], idx_v)
            copies = [
                pltpu.async_copy(tbl_ref.at[idx_v.at[j]],
                                 rows_v.at[pl.ds(j * 128, 128)], sem)
                for j in range(8)
            ]
            for cp in copies:
                cp.wait()
            out0 = pl.multiple_of(g * 1024, 8)
            pltpu.sync_copy(rows_v, out_ref.at[pl.ds(out0, 1024)])


def _gather_rows(d, tbl, win2d):
    k = pl.kernel(
        _gather_rows_body,
        out_type=jax.ShapeDtypeStruct((NP_, d), jnp.float32),
        mesh=_MESH,
        compiler_params=pltpu.CompilerParams(use_tc_tiling_on_sc=False),
        scratch_types=[pltpu.VMEM((8, 128), jnp.int32),
                       pltpu.VMEM((1024, d), jnp.float32),
                       pltpu.SemaphoreType.DMA],
    )
    return k(tbl, win2d)


def _edge_pass(src2d_ref, dst2d_ref, tbl_ref, acc, idx_s, idx_d, rows_v, sem,
               tile_row0):
    def chunk(k, _):
        row0 = pl.multiple_of(tile_row0 + k * (ECH // 128), 8)
        pltpu.sync_copy(src2d_ref.at[pl.ds(row0, ECH // 128)], idx_s)
        pltpu.sync_copy(dst2d_ref.at[pl.ds(row0, ECH // 128)], idx_d)
        for b in range(ECH // 128 // SUB):
            copies = [
                pltpu.async_copy(tbl_ref.at[idx_s.at[b * SUB + j]],
                                 rows_v.at[pl.ds(j * 128, 128)], sem)
                for j in range(SUB)
            ]
            for cp in copies:
                cp.wait()
            for j in range(SUB):
                pltpu.sync_copy(rows_v.at[pl.ds(j * 128, 128)],
                                acc.at[idx_d.at[b * SUB + j]], add=True)
        return 0

    lax.fori_loop(0, NCHUNK, chunk, 0)


def _zero_and_run(zeros_ref, acc, s, body):
    pltpu.sync_copy(zeros_ref,
                    acc.at[pl.ds(pl.multiple_of(s * TSTRIPE, 8), TSTRIPE)])
    plsc.subcore_barrier()
    body()
    plsc.subcore_barrier()


def _seg_body(dsrc_ref, ddst_ref, csrc_ref, cdst_ref, eri_ref,
              plo_ref, phi_ref, glo_ref, ghi_ref, elo_ref, ehi_ref,
              zeros_ref,
              alo_ref, ahi_ref, blo_ref, bhi_ref,
              acc, idx_s, idx_d, rows_v, sem):
    c = lax.axis_index("c")
    s = lax.axis_index("s")
    tile_row0 = s * ((EDP // 16) // 128)
    stripe = pl.ds(pl.multiple_of(s * TSTRIPE, 8), TSTRIPE)

    def half(p_ref, g_ref, e_ref, a_out, b_out):
        _zero_and_run(zeros_ref, acc, s, lambda: _edge_pass(
            dsrc_ref, ddst_ref, p_ref, acc, idx_s, idx_d, rows_v, sem,
            tile_row0))
        pltpu.sync_copy(acc.at[stripe], a_out.at[stripe])
        plsc.subcore_barrier()
        _zero_and_run(zeros_ref, acc, s, lambda: (
            _edge_pass(csrc_ref, cdst_ref, g_ref, acc, idx_s, idx_d, rows_v,
                       sem, tile_row0),
            _edge_pass(eri_ref, cdst_ref, e_ref, acc, idx_s, idx_d, rows_v,
                       sem, tile_row0),
        ))
        pltpu.sync_copy(acc.at[stripe], b_out.at[stripe])

    @pl.when(c == 0)
    def _():
        half(plo_ref, glo_ref, elo_ref, alo_ref, blo_ref)

    @pl.when(c == 1)
    def _():
        half(phi_ref, ghi_ref, ehi_ref, ahi_ref, bhi_ref)


def _seg_sums(dsrc2d, ddst2d, csrc2d, cdst2d, eri2d,
              p_lo, p_hi, g_lo, g_hi, e_lo, e_hi, zeros32):
    k = pl.kernel(
        _seg_body,
        out_type=[jax.ShapeDtypeStruct((NP_, 32), jnp.float32)] * 4,
        mesh=_MESH,
        compiler_params=pltpu.CompilerParams(use_tc_tiling_on_sc=False),
        scratch_types=[
            pltpu.VMEM_SHARED((NP_, 32), jnp.float32),
            pltpu.VMEM((ECH // 128, 128), jnp.int32),
            pltpu.VMEM((ECH // 128, 128), jnp.int32),
            pltpu.VMEM((SUB * 128, 32), jnp.float32),
            pltpu.SemaphoreType.DMA,
        ],
    )
    return k(dsrc2d, ddst2d, csrc2d, cdst2d, eri2d,
             p_lo, p_hi, g_lo, g_hi, e_lo, e_hi, zeros32)


def _count_body(ddst_ref, cdst_ref, ones_ref, zeros_ref, cntd_ref, cntc_ref,
                acc, idx_d, ones_v):
    c = lax.axis_index("c")
    s = lax.axis_index("s")
    tile_row0 = s * ((EDP // 16) // 128)
    stripe = pl.ds(pl.multiple_of(s * TSTRIPE, 8), TSTRIPE)
    pltpu.sync_copy(ones_ref, ones_v)

    def count(dst2d_ref, out_ref):
        pltpu.sync_copy(zeros_ref, acc.at[stripe])
        plsc.subcore_barrier()

        def chunk(k, _):
            row0 = pl.multiple_of(tile_row0 + k * (CCH // 128), 8)
            pltpu.sync_copy(dst2d_ref.at[pl.ds(row0, CCH // 128)], idx_d)
            for j in range(CCH // 128):
                pltpu.sync_copy(ones_v, acc.at[idx_d.at[j]], add=True)
            return 0

        lax.fori_loop(0, NCCHUNK, chunk, 0)
        plsc.subcore_barrier()
        pltpu.sync_copy(acc.at[stripe], out_ref.at[stripe])

    @pl.when(c == 0)
    def _():
        count(ddst_ref, cntd_ref)

    @pl.when(c == 1)
    def _():
        count(cdst_ref, cntc_ref)


def _counts(ddst2d, cdst2d, ones16, zeros16):
    k = pl.kernel(
        _count_body,
        out_type=[jax.ShapeDtypeStruct((NP_, 16), jnp.float32)] * 2,
        mesh=_MESH,
        compiler_params=pltpu.CompilerParams(use_tc_tiling_on_sc=False),
        scratch_types=[
            pltpu.VMEM_SHARED((NP_, 16), jnp.float32),
            pltpu.VMEM((CCH // 128, 128), jnp.int32),
            pltpu.VMEM((128, 16), jnp.float32),
        ],
    )
    return k(ddst2d, cdst2d, ones16, zeros16)


def kernel(reg_degree, gate_is_input, dev_edge_index, circ_edge_index,
           qubit_physical_idx, edge_reg_indices,
           reg_table, gate_table,
           W_self0, W_neigh0, b_sage0, W_self1, W_neigh1, b_sage1,
           W_circ0, b_circ0, W_circ1, b_circ1):
    f32 = jnp.float32
    deg2d = jnp.pad(reg_degree, (0, NP_ - NR)).reshape(NP_, 1)
    gii2d = jnp.pad(gate_is_input, (0, NP_ - NG)).reshape(NP_, 1)
    pad_e = EDP - ED
    dsrc2d = jnp.pad(dev_edge_index[0], (0, pad_e)).reshape(EDP // 128, 128)
    ddst2d = jnp.pad(dev_edge_index[1], (0, pad_e),
                     constant_values=NP_ - 1).reshape(EDP // 128, 128)
    csrc2d = jnp.pad(circ_edge_index[0], (0, pad_e)).reshape(EDP // 128, 128)
    cdst2d = jnp.pad(circ_edge_index[1], (0, pad_e),
                     constant_values=NP_ - 1).reshape(EDP // 128, 128)
    eri2d = jnp.pad(edge_reg_indices, (0, pad_e)).reshape(EDP // 128, 128)
    zeros32 = jnp.zeros((TSTRIPE, 32), f32)
    zeros16 = jnp.zeros((TSTRIPE, 16), f32)
    ones16 = jnp.ones((128, 16), f32)
    win = jnp.full((NR,), -1, jnp.int32).at[qubit_physical_idx].max(
        jnp.arange(NQ, dtype=jnp.int32))
    win = jnp.where(win < 0, NG, win)
    win2d = jnp.pad(win, (0, NP_ - NR),
                    constant_values=NG).reshape(NP_ // 128, 128)
    bs0 = b_sage0.reshape(1, 64)
    bs1 = b_sage1.reshape(1, 64)
    bc0 = b_circ0.reshape(1, 64)
    bc1 = b_circ1.reshape(1, 64)

    cnt_d, cnt_c = _counts(ddst2d, cdst2d, ones16, zeros16)
    reg_emb, gate_emb = _embed(deg2d, gii2d, reg_table, gate_table)

    sage = [(W_self0, W_neigh0, bs0), (W_self1, W_neigh1, bs1)]
    circ = [(W_circ0, bc0), (W_circ1, bc1)]
    d = 32
    for layer in range(2):
        ws, wn, bs = sage[layer]
        wc, bc = circ[layer]
        matched = _gather_rows(d, gate_emb, win2d)
        s, p_lo, p_hi, g_lo, g_hi, e_lo, e_hi = _prep(
            d, reg_emb, matched, gate_emb, ws, wn, wc)
        a_lo, a_hi, b_lo, b_hi = _seg_sums(
            dsrc2d, ddst2d, csrc2d, cdst2d, eri2d,
            p_lo, p_hi, g_lo, g_hi, e_lo, e_hi, zeros32)
        reg_emb, gate_emb = _epilogue(
            s, a_lo, a_hi, cnt_d, bs, b_lo, b_hi, cnt_c, bc)
        d = 64

    matched = _gather_rows(64, gate_emb, win2d)
    return jnp.concatenate([reg_emb[:NR], matched[:NR]], axis=1)


# speedup vs baseline: 3.0742x; 1.1745x over previous
import functools

import jax
import jax.numpy as jnp
from jax import lax
from jax.experimental import pallas as pl
from jax.experimental.pallas import tpu as pltpu
from jax.experimental.pallas import tpu_sc as plsc

NR = 50000
NG = 50000
ED = 800000
EC = 800000
NQ = 4096
DEG_TYPES = 16

NP_ = 50176
EDP = 819200
BLK = 1024
NBLK = NP_ // BLK
TSTRIPE = NP_ // 16
ECH = 1024
NBODY = (EDP // 16) // (2 * ECH)
TRASH = NP_ - 1

_MESH = plsc.VectorSubcoreMesh(core_axis_name="c", subcore_axis_name="s")
_SC_PARAMS = pltpu.CompilerParams(use_tc_tiling_on_sc=False)


def _leaky(x):
    return jnp.where(x >= 0, x, 0.01 * x)


def _n64(): return pl.BlockSpec((BLK, 64), lambda i: (i, 0))
def _n32(): return pl.BlockSpec((BLK, 32), lambda i: (i, 0))
def _bias(): return pl.BlockSpec((1, 64), lambda i: (0, 0))


def _dot(a, b):
    return jnp.dot(a, b, preferred_element_type=jnp.float32)


def _products(d, re_lo, re_hi, ge_lo, ge_hi, ws, wn, wc, out_refs):
    (rs_ref, rnl, rnh, qnl, qnh, qsl, qsh, gl, gh, el, eh) = out_refs

    def mm(x_lo, x_hi, w):
        if x_hi is None:
            return _dot(x_lo, w[:32])
        return _dot(x_lo, w[:32]) + _dot(x_hi, w[32:64])

    rs = mm(re_lo, re_hi, ws[:d])
    rn = mm(re_lo, re_hi, wn[:d])
    qn = mm(ge_lo, ge_hi, wn[d:])
    qs = mm(ge_lo, ge_hi, ws[d:])
    g = mm(ge_lo, ge_hi, wc[:d])
    e = mm(re_lo, re_hi, wc[d:])
    rs_ref[...] = rs
    rnl[...] = rn[:, :32]
    rnh[...] = rn[:, 32:]
    qnl[...] = qn[:, :32]
    qnh[...] = qn[:, 32:]
    qsl[...] = qs[:, :32]
    qsh[...] = qs[:, 32:]
    gl[...] = g[:, :32]
    gh[...] = g[:, 32:]
    el[...] = e[:, :32]
    eh[...] = e[:, 32:]


_PROD_OUT = ([jax.ShapeDtypeStruct((NP_, 64), jnp.float32)]
             + [jax.ShapeDtypeStruct((NP_, 32), jnp.float32)] * 10)


def _tc1_body(deg_ref, gii_ref, rtab_ref, gtab_ref, ws_ref, wn_ref, wc_ref,
              *out_refs):
    i = pl.program_id(0)
    deg = deg_ref[...]
    iota16 = lax.broadcasted_iota(jnp.int32, (BLK, DEG_TYPES), 1)
    onehot = (deg == iota16).astype(jnp.float32)
    re = _dot(onehot, rtab_ref[...])
    row0 = gtab_ref[0:1, :]
    row1 = gtab_ref[1:2, :]
    ge = jnp.where(gii_ref[...] == 1, row1, row0)
    rid = i * BLK + lax.broadcasted_iota(jnp.int32, (BLK, 1), 0)
    ge = jnp.where(rid < NG, ge, 0.0)
    _products(32, re, None, ge, None, ws_ref[...], wn_ref[...], wc_ref[...],
              out_refs)


def _tc1(deg2d, gii2d, rtab, gtab, ws0, wn0, wc0):
    return pl.pallas_call(
        _tc1_body,
        grid=(NBLK,),
        in_specs=[
            pl.BlockSpec((BLK, 1), lambda i: (i, 0)),
            pl.BlockSpec((BLK, 1), lambda i: (i, 0)),
            pl.BlockSpec((DEG_TYPES, 32), lambda i: (0, 0)),
            pl.BlockSpec((2, 32), lambda i: (0, 0)),
            pl.BlockSpec((64, 64), lambda i: (0, 0)),
            pl.BlockSpec((64, 64), lambda i: (0, 0)),
            pl.BlockSpec((64, 64), lambda i: (0, 0)),
        ],
        out_specs=[_n64()] + [_n32()] * 10,
        out_shape=_PROD_OUT,
    )(deg2d, gii2d, rtab, gtab, ws0, wn0, wc0)


def _epi_halves(rs_ref, msl, msh, al, ah, cntd, bs_ref,
                bl, bh, cntc, bc_ref):
    inv_d = 1.0 / jnp.maximum(cntd[:, 0:1], 1.0)
    cc = cntc[:, 0:1]
    inv_c = 1.0 / jnp.maximum(cc, 1.0)
    on = (cc > 0).astype(jnp.float32)
    rs = rs_ref[...]
    bs = bs_ref[...]
    bc = bc_ref[...]
    nr_lo = _leaky(rs[:, :32] + msl[...] + al[...] * inv_d + bs[:, :32])
    nr_hi = _leaky(rs[:, 32:] + msh[...] + ah[...] * inv_d + bs[:, 32:])
    ng_lo = _leaky(bl[...] * inv_c + bc[:, :32] * on)
    ng_hi = _leaky(bh[...] * inv_c + bc[:, 32:] * on)
    return nr_lo, nr_hi, ng_lo, ng_hi


def _tc2_body(rs_ref, msl, msh, al, ah, cntd, bs_ref, bl, bh, cntc, bc_ref,
              ws_ref, wn_ref, wc_ref, *out_refs):
    nr_lo, nr_hi, ng_lo, ng_hi = _epi_halves(
        rs_ref, msl, msh, al, ah, cntd, bs_ref, bl, bh, cntc, bc_ref)
    _products(64, nr_lo, nr_hi, ng_lo, ng_hi,
              ws_ref[...], wn_ref[...], wc_ref[...], out_refs)


def _tc2(rs, ms_lo, ms_hi, a_lo, a_hi, cntd32, bs2d, b_lo, b_hi, cntc32,
         bc2d, ws1, wn1, wc1):
    cnt = lambda: _n32()
    w = lambda: pl.BlockSpec((128, 64), lambda i: (0, 0))
    return pl.pallas_call(
        _tc2_body,
        grid=(NBLK,),
        in_specs=[_n64(), _n32(), _n32(), _n32(), _n32(), cnt(), _bias(),
                  _n32(), _n32(), cnt(), _bias(), w(), w(), w()],
        out_specs=[_n64()] + [_n32()] * 10,
        out_shape=_PROD_OUT,
    )(rs, ms_lo, ms_hi, a_lo, a_hi, cntd32, bs2d, b_lo, b_hi, cntc32, bc2d,
      ws1, wn1, wc1)


def _tc3_body(rs_ref, msl, msh, al, ah, cntd, bs_ref, bl, bh, cntc, bc_ref,
              nr_ref, ng_ref):
    nr_lo, nr_hi, ng_lo, ng_hi = _epi_halves(
        rs_ref, msl, msh, al, ah, cntd, bs_ref, bl, bh, cntc, bc_ref)
    nr_ref[:, :32] = nr_lo
    nr_ref[:, 32:] = nr_hi
    ng_ref[:, :32] = ng_lo
    ng_ref[:, 32:] = ng_hi


def _tc3(rs, ms_lo, ms_hi, a_lo, a_hi, cntd32, bs2d, b_lo, b_hi, cntc32,
         bc2d):
    cnt = lambda: _n32()
    return pl.pallas_call(
        _tc3_body,
        grid=(NBLK,),
        in_specs=[_n64(), _n32(), _n32(), _n32(), _n32(), cnt(), _bias(),
                  _n32(), _n32(), cnt(), _bias()],
        out_specs=[_n64(), _n64()],
        out_shape=[jax.ShapeDtypeStruct((NP_, 64), jnp.float32)] * 2,
    )(rs, ms_lo, ms_hi, a_lo, a_hi, cntd32, bs2d, b_lo, b_hi, cntc32, bc2d)


def _wait_like(src, dst, sem):
    pltpu.make_async_copy(src, dst, sem).wait()


def _edge_pass(srcs, dst2d, tbls, acc, ibufs, rbufs, sems, tile_row0):
    sem_i, sem_g, sem_s = sems
    (ia0, ib0, id0), (ia1, ib1, id1) = ibufs
    two_src = len(srcs) > 1
    two_tbl = len(tbls) > 1

    def stage(k, bufs, si):
        row0 = pl.multiple_of(tile_row0 + k * 8, 8)
        ia, ib, idd = bufs
        cps = [pltpu.async_copy(srcs[0].at[pl.ds(row0, 8)], ia, si),
               pltpu.async_copy(dst2d.at[pl.ds(row0, 8)], idd, si)]
        if two_src:
            cps.append(pltpu.async_copy(srcs[1].at[pl.ds(row0, 8)], ib, si))
        return cps

    def wait_stage(k, bufs, si):
        row0 = pl.multiple_of(tile_row0 + k * 8, 8)
        ia, ib, idd = bufs
        _wait_like(srcs[0].at[pl.ds(row0, 8)], ia, si)
        _wait_like(dst2d.at[pl.ds(row0, 8)], idd, si)
        if two_src:
            _wait_like(srcs[1].at[pl.ds(row0, 8)], ib, si)

    stage(0, (ia0, ib0, id0), sem_i[0])
    stage(1, (ia1, ib1, id1), sem_i[1])

    def body(k, _):
        wait_stage(2 * k, (ia0, ib0, id0), sem_i[0])
        wait_stage(2 * k + 1, (ia1, ib1, id1), sem_i[1])
        pend_g = [None, None]
        pend_s = [None, None]
        sets = ((ia0, ib0, id0), (ia1, ib1, id1))

        def fire_gather(j, p):
            ia, ib, idd = sets[j // 8]
            jj = j % 8
            r = rbufs[p]
            cps = [pltpu.async_copy(tbls[0].at[ia.at[jj]],
                                    r.at[pl.ds(0, 128)], sem_g[p])]
            src2 = (ib if two_src else ia)
            if two_tbl:
                cps.append(pltpu.async_copy(tbls[1].at[src2.at[jj]],
                                            r.at[pl.ds(128, 128)],
                                            sem_g[p]))
            pend_g[p] = cps

        def fire_scatter(j, p):
            _, _, idd = sets[j // 8]
            jj = j % 8
            r = rbufs[p]
            cps = [pltpu.async_copy(r.at[pl.ds(0, 128)], acc.at[idd.at[jj]],
                                    sem_s[p], add=True)]
            if two_tbl:
                cps.append(pltpu.async_copy(r.at[pl.ds(128, 128)],
                                            acc.at[idd.at[jj]], sem_s[p],
                                            add=True))
            pend_s[p] = cps

        for j in range(16):
            p = j % 2
            if j >= 2:
                for cp in pend_s[p]:
                    cp.wait()
            fire_gather(j, p)
            if j >= 1:
                for cp in pend_g[1 - p]:
                    cp.wait()
                fire_scatter(j - 1, 1 - p)
            if j == 10:
                @pl.when(k < NBODY - 1)
                def _():
                    stage(2 * k + 2, (ia0, ib0, id0), sem_i[0])
        for cp in pend_g[1]:
            cp.wait()
        fire_scatter(15, 1)
        for cp in pend_s[0]:
            cp.wait()
        for cp in pend_s[1]:
            cp.wait()

        @pl.when(k < NBODY - 1)
        def _():
            stage(2 * k + 3, (ia1, ib1, id1), sem_i[1])

        return 0

    lax.fori_loop(0, NBODY, body, 0)


def _node_gather(tbl_a, tbl_b, win2d, out_a, out_b, ibuf, rbufs, sems, tid):
    sem_i, sem_g, sem_s = sems
    for r in range((NBLK + 15) // 16):
        g = r * 16 + tid

        @pl.when(g < NBLK)
        def _():
            row0 = pl.multiple_of(g * 8, 8)
            pltpu.sync_copy(win2d.at[pl.ds(row0, 8)], ibuf)
            pend_g = [None, None]
            pend_s = [None, None]

            def fire_gather(j, p):
                rb = rbufs[p]
                pend_g[p] = [
                    pltpu.async_copy(tbl_a.at[ibuf.at[j]],
                                     rb.at[pl.ds(0, 128)], sem_g[p]),
                    pltpu.async_copy(tbl_b.at[ibuf.at[j]],
                                     rb.at[pl.ds(128, 128)], sem_g[p]),
                ]

            def fire_out(j, p):
                rb = rbufs[p]
                o0 = pl.multiple_of(g * 1024 + j * 128, 8)
                pend_s[p] = [
                    pltpu.async_copy(rb.at[pl.ds(0, 128)],
                                     out_a.at[pl.ds(o0, 128)], sem_s[p]),
                    pltpu.async_copy(rb.at[pl.ds(128, 128)],
                                     out_b.at[pl.ds(o0, 128)], sem_s[p]),
                ]

            for j in range(8):
                p = j % 2
                if j >= 2:
                    for cp in pend_s[p]:
                        cp.wait()
                fire_gather(j, p)
                if j >= 1:
                    for cp in pend_g[1 - p]:
                        cp.wait()
                    fire_out(j - 1, 1 - p)
            for cp in pend_g[1]:
                cp.wait()
            fire_out(7, 1)
            for cp in pend_s[0]:
                cp.wait()
            for cp in pend_s[1]:
                cp.wait()


def _count_pass(dst2d, ones_v, acc, ibufs, sems, tile_row0):
    sem_i, sem_g, sem_s = sems
    (_, _, id0), (_, _, id1) = ibufs

    def stage(k, idd, si):
        row0 = pl.multiple_of(tile_row0 + k * 8, 8)
        pltpu.async_copy(dst2d.at[pl.ds(row0, 8)], idd, si)

    def wait_stage(k, idd, si):
        row0 = pl.multiple_of(tile_row0 + k * 8, 8)
        _wait_like(dst2d.at[pl.ds(row0, 8)], idd, si)

    stage(0, id0, sem_i[0])
    stage(1, id1, sem_i[1])

    def body(k, _):
        for half, idd, si, ss in ((0, id0, sem_i[0], sem_s[0]),
                                  (1, id1, sem_i[1], sem_s[1])):
            wait_stage(2 * k + half, idd, si)
            cps = [pltpu.async_copy(ones_v, acc.at[idd.at[j]], ss,
                                    add=True)
                   for j in range(8)]
            for cp in cps:
                cp.wait()

            @pl.when(k < NBODY - 1)
            def _():
                stage(2 * k + 2 + half, idd, si)
        return 0

    lax.fori_loop(0, NBODY, body, 0)


def _sc_layer_body(first_layer, dsrc2d_ref, ddst2d_ref, csrc2d_ref,
                   cdst2d_ref, eri2d_ref, win2d_ref,
                   rnl_ref, rnh_ref, qnl_ref, qnh_ref, qsl_ref, qsh_ref,
                   gl_ref, gh_ref, el_ref, eh_ref, zeros_ref, ones_ref,
                   msl_ref, msh_ref, mnl_ref, mnh_ref,
                   al_ref, ah_ref, bl_ref, bh_ref, cntd_ref, cntc_ref,
                   acc, ia0, ib0, id0, ia1, ib1, id1, rb0, rb1, ones_v,
                   sem_i0, sem_i1, sem_g0, sem_g1, sem_s0, sem_s1):
    c = lax.axis_index("c")
    s = lax.axis_index("s")
    wid = s * 2 + c
    tile_row0 = s * ((EDP // 16) // 128)
    stripe = pl.ds(pl.multiple_of(s * TSTRIPE, 8), TSTRIPE)
    sems = ((sem_i0, sem_i1), (sem_g0, sem_g1), (sem_s0, sem_s1))
    ibufs = ((ia0, ib0, id0), (ia1, ib1, id1))
    rbufs = (rb0, rb1)

    def zero_acc():
        pltpu.sync_copy(zeros_ref, acc.at[stripe])

    def run_half(qn_ref, qs_ref, mn_ref, ms_ref, rn_ref, g_ref, e_ref,
                 a_ref, b_ref, cnt_dst2d, cnt_out):
        _node_gather(qn_ref, qs_ref, win2d_ref, mn_ref, ms_ref, ia0, rbufs,
                     sems, s)
        if first_layer:
            pltpu.sync_copy(ones_ref, ones_v)
            zero_acc()
            plsc.subcore_barrier()
            _count_pass(cnt_dst2d, ones_v, acc, ibufs, sems, tile_row0)
            plsc.subcore_barrier()
            pltpu.sync_copy(acc.at[stripe], cnt_out.at[stripe])
        zero_acc()
        plsc.subcore_barrier()
        _edge_pass([dsrc2d_ref], ddst2d_ref, [rn_ref, mn_ref], acc, ibufs,
                   rbufs, sems, tile_row0)
        plsc.subcore_barrier()
        pltpu.sync_copy(acc.at[stripe], a_ref.at[stripe])
        zero_acc()
        plsc.subcore_barrier()
        _edge_pass([csrc2d_ref, eri2d_ref], cdst2d_ref, [g_ref, e_ref], acc,
                   ibufs, rbufs, sems, tile_row0)
        plsc.subcore_barrier()
        pltpu.sync_copy(acc.at[stripe], b_ref.at[stripe])

    @pl.when(c == 0)
    def _():
        run_half(qnl_ref, qsl_ref, mnl_ref, msl_ref, rnl_ref, gl_ref, el_ref,
                 al_ref, bl_ref, ddst2d_ref, cntd_ref)

    @pl.when(c == 1)
    def _():
        run_half(qnh_ref, qsh_ref, mnh_ref, msh_ref, rnh_ref, gh_ref, eh_ref,
                 ah_ref, bh_ref, cdst2d_ref, cntc_ref)


def _sc_layer(first_layer, dsrc2d, ddst2d, csrc2d, cdst2d, eri2d, win2d,
              rn_lo, rn_hi, qn_lo, qn_hi, qs_lo, qs_hi, g_lo, g_hi, e_lo,
              e_hi, zeros32, ones32):
    n_out = 10 if first_layer else 8
    if first_layer:
        body = functools.partial(_sc_layer_body, first_layer)
    else:
        def body(*args):
            ins = args[:18]
            outs = args[18:26]
            scr = args[26:]
            return _sc_layer_body(False, *ins, *outs, None, None, *scr)
    k = pl.kernel(
        body,
        out_type=[jax.ShapeDtypeStruct((NP_, 32), jnp.float32)] * n_out,
        mesh=_MESH,
        compiler_params=_SC_PARAMS,
        scratch_types=[
            pltpu.VMEM_SHARED((NP_, 32), jnp.float32),
            pltpu.VMEM((8, 128), jnp.int32),
            pltpu.VMEM((8, 128), jnp.int32),
            pltpu.VMEM((8, 128), jnp.int32),
            pltpu.VMEM((8, 128), jnp.int32),
            pltpu.VMEM((8, 128), jnp.int32),
            pltpu.VMEM((8, 128), jnp.int32),
            pltpu.VMEM((256, 32), jnp.float32),
            pltpu.VMEM((256, 32), jnp.float32),
            pltpu.VMEM((128, 32), jnp.float32),
            pltpu.SemaphoreType.DMA,
            pltpu.SemaphoreType.DMA,
            pltpu.SemaphoreType.DMA,
            pltpu.SemaphoreType.DMA,
            pltpu.SemaphoreType.DMA,
            pltpu.SemaphoreType.DMA,
        ],
    )
    return k(dsrc2d, ddst2d, csrc2d, cdst2d, eri2d, win2d,
             rn_lo, rn_hi, qn_lo, qn_hi, qs_lo, qs_hi, g_lo, g_hi, e_lo,
             e_hi, zeros32, ones32)


def _final_gather_body(tbl_ref, win2d_ref, out_ref, idx_v, rows_v, sem):
    c = lax.axis_index("c")
    s = lax.axis_index("s")
    wid = s * 2 + c
    for rep in range((NBLK + 31) // 32):
        g = rep * 32 + wid

        @pl.when(g < NBLK)
        def _():
            row0 = pl.multiple_of(g * 8, 8)
            pltpu.sync_copy(win2d_ref.at[pl.ds(row0, 8)], idx_v)
            copies = [
                pltpu.async_copy(tbl_ref.at[idx_v.at[j]],
                                 rows_v.at[pl.ds(j * 128, 128)], sem)
                for j in range(8)
            ]
            for cp in copies:
                cp.wait()
            out0 = pl.multiple_of(g * 1024, 8)
            pltpu.sync_copy(rows_v, out_ref.at[pl.ds(out0, 1024)])


def _final_gather(tbl, win2d):
    k = pl.kernel(
        _final_gather_body,
        out_type=jax.ShapeDtypeStruct((NP_, 64), jnp.float32),
        mesh=_MESH,
        compiler_params=_SC_PARAMS,
        scratch_types=[pltpu.VMEM((8, 128), jnp.int32),
                       pltpu.VMEM((1024, 64), jnp.float32),
                       pltpu.SemaphoreType.DMA],
    )
    return k(tbl, win2d)


def kernel(reg_degree, gate_is_input, dev_edge_index, circ_edge_index,
           qubit_physical_idx, edge_reg_indices,
           reg_table, gate_table,
           W_self0, W_neigh0, b_sage0, W_self1, W_neigh1, b_sage1,
           W_circ0, b_circ0, W_circ1, b_circ1):
    f32 = jnp.float32
    deg2d = jnp.pad(reg_degree, (0, NP_ - NR)).reshape(NP_, 1)
    gii2d = jnp.pad(gate_is_input, (0, NP_ - NG)).reshape(NP_, 1)
    pad_e = EDP - ED
    dsrc2d = jnp.pad(dev_edge_index[0], (0, pad_e)).reshape(EDP // 128, 128)
    ddst2d = jnp.pad(dev_edge_index[1], (0, pad_e),
                     constant_values=TRASH).reshape(EDP // 128, 128)
    csrc2d = jnp.pad(circ_edge_index[0], (0, pad_e)).reshape(EDP // 128, 128)
    cdst2d = jnp.pad(circ_edge_index[1], (0, pad_e),
                     constant_values=TRASH).reshape(EDP // 128, 128)
    eri2d = jnp.pad(edge_reg_indices, (0, pad_e)).reshape(EDP // 128, 128)
    zeros32 = jnp.zeros((TSTRIPE, 32), f32)
    ones32 = jnp.ones((128, 32), f32)
    win = jnp.full((NR,), -1, jnp.int32).at[qubit_physical_idx].max(
        jnp.arange(NQ, dtype=jnp.int32))
    win = jnp.where(win < 0, NG, win)
    win2d = jnp.pad(win, (0, NP_ - NR),
                    constant_values=NG).reshape(NP_ // 128, 128)
    bs0 = b_sage0.reshape(1, 64)
    bs1 = b_sage1.reshape(1, 64)
    bc0 = b_circ0.reshape(1, 64)
    bc1 = b_circ1.reshape(1, 64)

    prods = _tc1(deg2d, gii2d, reg_table, gate_table, W_self0, W_neigh0,
                 W_circ0)
    rs, rnl, rnh, qnl, qnh, qsl, qsh, gl, gh, el, eh = prods
    (msl, msh, mnl, mnh, al, ah, bl, bh, cntd32, cntc32) = _sc_layer(
        True, dsrc2d, ddst2d, csrc2d, cdst2d, eri2d, win2d,
        rnl, rnh, qnl, qnh, qsl, qsh, gl, gh, el, eh, zeros32, ones32)
    prods = _tc2(rs, msl, msh, al, ah, cntd32, bs0, bl, bh, cntc32, bc0,
                 W_self1, W_neigh1, W_circ1)
    rs, rnl, rnh, qnl, qnh, qsl, qsh, gl, gh, el, eh = prods
    (msl, msh, mnl, mnh, al, ah, bl, bh) = _sc_layer(
        False, dsrc2d, ddst2d, csrc2d, cdst2d, eri2d, win2d,
        rnl, rnh, qnl, qnh, qsl, qsh, gl, gh, el, eh, zeros32, ones32)
    nr2, ng2 = _tc3(rs, msl, msh, al, ah, cntd32, bs1, bl, bh, cntc32, bc1)
    matched = _final_gather(ng2, win2d)
    return jnp.concatenate([nr2[:NR], matched[:NR]], axis=1)
